# baseline (speedup 1.0000x reference)
import functools

import jax
import jax.numpy as jnp
from jax import lax
from jax.experimental import pallas as pl
from jax.experimental.pallas import tpu as pltpu
from jax.experimental.pallas import tpu_sc as plsc

N_ATOMS = 50000
N_CONF = 128
N_PAD = 50176
CHUNK = N_PAD // 32
NVREG = CHUNK // 16
LANE = 16
ROW_F = N_CONF * 3
SB = 112
NB = CHUNK // SB
LAST_FULL = (N_ATOMS - 31 * CHUNK) // SB
LAST_REM = N_ATOMS - 31 * CHUNK - LAST_FULL * SB


def _body(sp_hbm, ct_hbm, ss_hbm, out_hbm,
          mine_v, other_v, posbuf, postail, ssbuf, cnt_stage, hist_v,
          data_a, data_b, hist_sh, sem_a, sem_b):
    cid = lax.axis_index("c")
    tid = lax.axis_index("s")
    lane = lax.iota(jnp.int32, LANE)
    full15 = jnp.full((LANE,), 15, jnp.int32)
    w = cid * 16 + tid
    w_other = (1 - cid) * 16 + tid
    base = w * CHUNK
    pltpu.sync_copy(sp_hbm.at[pl.ds(base, CHUNK)], mine_v)
    pltpu.sync_copy(sp_hbm.at[pl.ds(w_other * CHUNK, CHUNK)], other_v)

    def hist(chunk_ref, slot):
        def step(i, accs):
            a01, a34 = accs
            x = chunk_ref[pl.ds(i * LANE, LANE)]
            sa = jnp.minimum(10 * x, 31)
            sb = jnp.clip(10 * (x - 3), 0, 31)
            a01 = a01 + jnp.where(x <= 2, jnp.left_shift(1, sa), 0)
            a34 = a34 + jnp.where(x >= 3, jnp.left_shift(1, sb), 0)
            return a01, a34

        a01, a34 = lax.fori_loop(0, NVREG, step,
                                 (jnp.zeros((LANE,), jnp.int32),
                                  jnp.zeros((LANE,), jnp.int32)))
        counts = jnp.zeros((LANE,), jnp.int32)
        for v in range(3):
            counts = counts + jnp.where(
                lane == v, jnp.sum((a01 >> (10 * v)) & 1023), 0)
        for v in range(3, 5):
            counts = counts + jnp.where(
                lane == v, jnp.sum((a34 >> (10 * (v - 3))) & 1023), 0)
        cnt_stage[...] = counts
        pltpu.sync_copy(cnt_stage, hist_sh.at[pl.ds(slot * LANE, LANE)])

    hist(mine_v, w)
    hist(other_v, w_other)
    plsc.subcore_barrier()
    pltpu.sync_copy(hist_sh, hist_v)

    def acc_step(t2, carry):
        totals, prefix = carry
        row = hist_v[pl.ds(t2 * LANE, LANE)]
        totals = totals + row
        prefix = prefix + jnp.where(t2 < w, row, 0)
        return totals, prefix

    totals, prefix = lax.fori_loop(
        0, 32, acc_step,
        (jnp.zeros((LANE,), jnp.int32), jnp.zeros((LANE,), jnp.int32)))
    starts = plsc.cumsum(totals) - totals
    offs0 = starts + prefix
    s_cls = [jnp.sum(jnp.where(lane == v, starts, 0)) for v in range(1, 5)]
    sh_lane = jnp.minimum(6 * lane, 31)

    def rank_step(i, offs):
        x = mine_v[pl.ds(i * LANE, LANE)]
        packed = plsc.cumsum(jnp.left_shift(1, 6 * x))
        myrank = (packed >> (6 * x)) & 63
        pos = offs.at[x].get(mode="promise_in_bounds") + myrank - 1
        last = packed.at[full15].get(mode="promise_in_bounds")
        offs = offs + jnp.where(lane < 5, (last >> sh_lane) & 63, 0)
        row = i // 7
        col = (i % 7) * LANE
        posbuf[row, pl.ds(col, LANE)] = pos

        @pl.when(jnp.logical_and(i >= 84, i <= 86))
        def _():
            postail[0, pl.ds((i - 84) * LANE, LANE)] = pos

        g = base + i * LANE + lane
        val = jnp.zeros((LANE,), jnp.int32)
        for sv in s_cls:
            val = val + jnp.where(g >= sv, 1, 0)
        ssbuf[pl.ds(i * LANE, LANE)] = val
        return offs

    lax.fori_loop(0, NVREG, rank_step, offs0)
    pltpu.sync_copy(ssbuf, ss_hbm.at[pl.ds(base, CHUNK)])

    def load(k, buf, sem):
        return pltpu.async_copy(ct_hbm.at[pl.ds(base + k * SB, SB)],
                                buf, sem)

    bufs = (data_a, data_b)
    sems = (sem_a, sem_b)

    def pipeline(nb):
        ld = load(0, bufs[0], sems[0])
        for k in range(nb):
            nxt = None
            if k + 1 < nb:
                nxt = load(k + 1, bufs[(k + 1) % 2], sems[(k + 1) % 2])
            ld.wait()
            pltpu.async_copy(bufs[k % 2], out_hbm.at[posbuf.at[k]],
                             sems[k % 2]).wait()
            ld = nxt

    @pl.when(w < 31)
    def _():
        pipeline(NB)

    @pl.when(w == 31)
    def _():
        pipeline(LAST_FULL)
        pltpu.sync_copy(ct_hbm.at[pl.ds(base + LAST_FULL * SB, LAST_REM)],
                        data_a.at[pl.ds(0, LAST_REM)])
        pltpu.async_copy(data_a.at[pl.ds(0, LAST_REM)],
                         out_hbm.at[postail.at[0]], sem_a).wait()


def kernel(species, coordinates):
    mesh = plsc.VectorSubcoreMesh(core_axis_name="c", subcore_axis_name="s")
    sp_pad = jnp.concatenate(
        [species, jnp.full((N_PAD - N_ATOMS,), 4, jnp.int32)])
    ct = jnp.transpose(coordinates, (1, 0, 2)).reshape(N_ATOMS, ROW_F)

    sc_k = functools.partial(
        pl.kernel,
        out_type=(jax.ShapeDtypeStruct((N_PAD,), jnp.int32),
                  jax.ShapeDtypeStruct((N_ATOMS, ROW_F), jnp.float32)),
        mesh=mesh,
        compiler_params=pltpu.CompilerParams(needs_layout_passes=False,
                                             use_tc_tiling_on_sc=False),
        scratch_types=[
            pltpu.VMEM((CHUNK,), jnp.int32),
            pltpu.VMEM((CHUNK,), jnp.int32),
            pltpu.VMEM((NB, SB), jnp.int32),
            pltpu.VMEM((1, LAST_REM), jnp.int32),
            pltpu.VMEM((CHUNK,), jnp.int32),
            pltpu.VMEM((LANE,), jnp.int32),
            pltpu.VMEM((32 * LANE,), jnp.int32),
            pltpu.VMEM((SB, ROW_F), jnp.float32),
            pltpu.VMEM((SB, ROW_F), jnp.float32),
            pltpu.VMEM_SHARED((32 * LANE,), jnp.int32),
            pltpu.SemaphoreType.DMA,
            pltpu.SemaphoreType.DMA,
        ],
    )(_body)
    ss, out_t = sc_k(sp_pad, ct)
    new_coords = jnp.transpose(out_t.reshape(N_ATOMS, N_CONF, 3), (1, 0, 2))
    return ss[:N_ATOMS], new_coords


# speedup vs baseline: 2.1007x; 2.1007x over previous
import functools

import jax
import jax.numpy as jnp
from jax import lax
from jax.experimental import pallas as pl
from jax.experimental.pallas import tpu as pltpu
from jax.experimental.pallas import tpu_sc as plsc

N_ATOMS = 50000
N_CONF = 128
N_PAD = 50176
CHUNK = N_PAD // 32
NVREG = CHUNK // 16
LANE = 16
ROW_F = N_CONF * 3
SB = 112
NB = CHUNK // SB
LAST_FULL = (N_ATOMS - 31 * CHUNK) // SB
LAST_REM = N_ATOMS - 31 * CHUNK - LAST_FULL * SB


def _body(sp_hbm, ct_hbm, ss_hbm, out_hbm,
          mine_v, other_v, posbuf, postail, ssbuf, cnt_stage, hist_v,
          data_a, data_b, hist_sh, sem_a, sem_b):
    cid = lax.axis_index("c")
    tid = lax.axis_index("s")
    lane = lax.iota(jnp.int32, LANE)
    full15 = jnp.full((LANE,), 15, jnp.int32)
    w = cid * 16 + tid
    w_other = (1 - cid) * 16 + tid
    base = w * CHUNK
    pltpu.sync_copy(sp_hbm.at[pl.ds(base, CHUNK)], mine_v)
    pltpu.sync_copy(sp_hbm.at[pl.ds(w_other * CHUNK, CHUNK)], other_v)

    def hist(chunk_ref, slot):
        def step(i, accs):
            a01, a34 = accs
            x = chunk_ref[pl.ds(i * LANE, LANE)]
            sa = jnp.minimum(10 * x, 31)
            sb = jnp.clip(10 * (x - 3), 0, 31)
            a01 = a01 + jnp.where(x <= 2, jnp.left_shift(1, sa), 0)
            a34 = a34 + jnp.where(x >= 3, jnp.left_shift(1, sb), 0)
            return a01, a34

        a01, a34 = lax.fori_loop(0, NVREG, step,
                                 (jnp.zeros((LANE,), jnp.int32),
                                  jnp.zeros((LANE,), jnp.int32)))
        counts = jnp.zeros((LANE,), jnp.int32)
        for v in range(3):
            counts = counts + jnp.where(
                lane == v, jnp.sum((a01 >> (10 * v)) & 1023), 0)
        for v in range(3, 5):
            counts = counts + jnp.where(
                lane == v, jnp.sum((a34 >> (10 * (v - 3))) & 1023), 0)
        cnt_stage[...] = counts
        pltpu.sync_copy(cnt_stage, hist_sh.at[pl.ds(slot * LANE, LANE)])

    hist(mine_v, w)
    hist(other_v, w_other)
    plsc.subcore_barrier()
    pltpu.sync_copy(hist_sh, hist_v)

    def acc_step(t2, carry):
        totals, prefix = carry
        row = hist_v[pl.ds(t2 * LANE, LANE)]
        totals = totals + row
        prefix = prefix + jnp.where(t2 < w, row, 0)
        return totals, prefix

    totals, prefix = lax.fori_loop(
        0, 32, acc_step,
        (jnp.zeros((LANE,), jnp.int32), jnp.zeros((LANE,), jnp.int32)))
    starts = plsc.cumsum(totals) - totals
    offs0 = starts + prefix
    s_cls = [jnp.sum(jnp.where(lane == v, starts, 0)) for v in range(1, 5)]
    sh_lane = jnp.minimum(6 * lane, 31)

    def rank_step(i, offs):
        x = mine_v[pl.ds(i * LANE, LANE)]
        packed = plsc.cumsum(jnp.left_shift(1, 6 * x))
        myrank = (packed >> (6 * x)) & 63
        pos = offs.at[x].get(mode="promise_in_bounds") + myrank - 1
        last = packed.at[full15].get(mode="promise_in_bounds")
        offs = offs + jnp.where(lane < 5, (last >> sh_lane) & 63, 0)
        row = i // 7
        col = (i % 7) * LANE
        posbuf[row, pl.ds(col, LANE)] = pos

        @pl.when(jnp.logical_and(i >= 84, i <= 86))
        def _():
            postail[0, pl.ds((i - 84) * LANE, LANE)] = pos

        g = base + i * LANE + lane
        val = jnp.zeros((LANE,), jnp.int32)
        for sv in s_cls:
            val = val + jnp.where(g >= sv, 1, 0)
        ssbuf[pl.ds(i * LANE, LANE)] = val
        return offs

    lax.fori_loop(0, NVREG, rank_step, offs0)
    pltpu.sync_copy(ssbuf, ss_hbm.at[pl.ds(base, CHUNK)])

    def load(k, buf, sem):
        return pltpu.async_copy(ct_hbm.at[pl.ds(base + k * SB, SB)],
                                buf, sem)

    bufs = (data_a, data_b)
    sems = (sem_a, sem_b)

    def pipeline(nb):
        ld = load(0, bufs[0], sems[0])
        for k in range(nb):
            nxt = None
            if k + 1 < nb:
                nxt = load(k + 1, bufs[(k + 1) % 2], sems[(k + 1) % 2])
            ld.wait()
            pltpu.async_copy(bufs[k % 2], out_hbm.at[posbuf.at[k]],
                             sems[k % 2]).wait()
            ld = nxt

    @pl.when(w < 31)
    def _():
        pipeline(NB)

    @pl.when(w == 31)
    def _():
        pipeline(LAST_FULL)
        pltpu.sync_copy(ct_hbm.at[pl.ds(base + LAST_FULL * SB, LAST_REM)],
                        data_a.at[pl.ds(0, LAST_REM)])
        pltpu.async_copy(data_a.at[pl.ds(0, LAST_REM)],
                         out_hbm.at[postail.at[0]], sem_a).wait()


def kernel(species, coordinates):
    mesh = plsc.VectorSubcoreMesh(core_axis_name="c", subcore_axis_name="s")
    sp_pad = jnp.concatenate(
        [species, jnp.full((N_PAD - N_ATOMS,), 4, jnp.int32)])
    ct = jnp.transpose(coordinates, (1, 2, 0)).reshape(N_ATOMS, ROW_F)

    sc_k = functools.partial(
        pl.kernel,
        out_type=(jax.ShapeDtypeStruct((N_PAD,), jnp.int32),
                  jax.ShapeDtypeStruct((N_ATOMS, ROW_F), jnp.float32)),
        mesh=mesh,
        compiler_params=pltpu.CompilerParams(needs_layout_passes=False),
        scratch_types=[
            pltpu.VMEM((CHUNK,), jnp.int32),
            pltpu.VMEM((CHUNK,), jnp.int32),
            pltpu.VMEM((NB, SB), jnp.int32),
            pltpu.VMEM((1, LAST_REM), jnp.int32),
            pltpu.VMEM((CHUNK,), jnp.int32),
            pltpu.VMEM((LANE,), jnp.int32),
            pltpu.VMEM((32 * LANE,), jnp.int32),
            pltpu.VMEM((SB, ROW_F), jnp.float32),
            pltpu.VMEM((SB, ROW_F), jnp.float32),
            pltpu.VMEM_SHARED((32 * LANE,), jnp.int32),
            pltpu.SemaphoreType.DMA,
            pltpu.SemaphoreType.DMA,
        ],
    )(_body)
    ss, out_t = sc_k(sp_pad, ct)
    new_coords = jnp.transpose(out_t.reshape(N_ATOMS, 3, N_CONF), (2, 0, 1))
    return ss[:N_ATOMS], new_coords
